# Initial kernel scaffold; baseline (speedup 1.0000x reference)
import functools

import jax
import jax.numpy as jnp
from jax import lax
from jax.experimental import pallas as pl
from jax.experimental.pallas import tpu as pltpu
from jax.experimental.pallas import tpu_sc as plsc

NC = 2
NS = 16
NW = NC * NS
CHUNK = 128


def _segment_sum_sc(p, src2d, dst2d, zblk, n_nodes, acc_rows, cpw):
    H = p.shape[1]
    sub_rows = acc_rows // NS
    mesh = plsc.VectorSubcoreMesh(core_axis_name="c", subcore_axis_name="s")
    last_rows = n_nodes - (NS - 1) * sub_rows

    @functools.partial(
        pl.kernel,
        out_type=jax.ShapeDtypeStruct((NC, n_nodes, H), jnp.float32),
        mesh=mesh,
        scratch_types=[
            pltpu.VMEM((cpw, CHUNK), jnp.int32),
            pltpu.VMEM((cpw, CHUNK), jnp.int32),
            pltpu.VMEM((CHUNK, H), jnp.float32),
            pltpu.VMEM_SHARED((acc_rows, H), jnp.float32),
        ],
    )
    def seg_kernel(p_hbm, s_hbm, d_hbm, z_hbm, out_hbm, sblk, dblk, rows, acc):
        c = lax.axis_index("c")
        s = lax.axis_index("s")
        w = c * NS + s

        pltpu.sync_copy(z_hbm, acc.at[pl.ds(s * sub_rows, sub_rows)])
        pltpu.sync_copy(s_hbm.at[pl.ds(w * cpw, cpw)], sblk)
        pltpu.sync_copy(d_hbm.at[pl.ds(w * cpw, cpw)], dblk)
        plsc.subcore_barrier()

        @pl.loop(0, cpw)
        def _(i):
            pltpu.sync_copy(p_hbm.at[sblk.at[i]], rows)
            pltpu.sync_copy(rows, acc.at[dblk.at[i]], add=True)

        plsc.subcore_barrier()

        @pl.when(s < NS - 1)
        def _():
            pltpu.sync_copy(acc.at[pl.ds(s * sub_rows, sub_rows)],
                            out_hbm.at[c, pl.ds(s * sub_rows, sub_rows)])

        @pl.when(s == NS - 1)
        def _():
            pltpu.sync_copy(acc.at[pl.ds((NS - 1) * sub_rows, last_rows)],
                            out_hbm.at[c, pl.ds((NS - 1) * sub_rows, last_rows)])

    return seg_kernel(p, src2d, dst2d, zblk)


def _proj_first(x, w, blk):
    n, d = x.shape
    h = w.shape[1]

    def body(x_ref, w_ref, o_ref):
        o_ref[...] = jnp.dot(x_ref[...], w_ref[...],
                             preferred_element_type=jnp.float32)

    return pl.pallas_call(
        body,
        grid=(n // blk,),
        in_specs=[pl.BlockSpec((blk, d), lambda i: (i, 0)),
                  pl.BlockSpec((d, h), lambda i: (0, 0))],
        out_specs=pl.BlockSpec((blk, h), lambda i: (i, 0)),
        out_shape=jax.ShapeDtypeStruct((n, h), jnp.float32),
    )(x, w)


def _mid_stage(p, agg, b1, m, c, blk):
    n, h = p.shape
    ho = m.shape[1]

    def body(p_ref, g_ref, b_ref, m_ref, c_ref, o_ref):
        u = jnp.maximum(p_ref[...] + g_ref[0] + g_ref[1] + b_ref[...], 0.0)
        o_ref[...] = jnp.dot(u, m_ref[...],
                             preferred_element_type=jnp.float32) + c_ref[...]

    return pl.pallas_call(
        body,
        grid=(n // blk,),
        in_specs=[pl.BlockSpec((blk, h), lambda i: (i, 0)),
                  pl.BlockSpec((NC, blk, h), lambda i: (0, i, 0)),
                  pl.BlockSpec((1, h), lambda i: (0, 0)),
                  pl.BlockSpec((h, ho), lambda i: (0, 0)),
                  pl.BlockSpec((1, ho), lambda i: (0, 0))],
        out_specs=pl.BlockSpec((blk, ho), lambda i: (i, 0)),
        out_shape=jax.ShapeDtypeStruct((n, ho), jnp.float32),
    )(p, agg, b1.reshape(1, h), m, c.reshape(1, ho))


def _final_stage(p, agg, b1, w2, b2, blk):
    n, h = p.shape

    def body(p_ref, g_ref, b_ref, w_ref, c_ref, o_ref):
        u = jnp.maximum(p_ref[...] + g_ref[0] + g_ref[1] + b_ref[...], 0.0)
        o_ref[...] = jnp.sum(u * w_ref[...], axis=1, keepdims=True) + c_ref[...]

    return pl.pallas_call(
        body,
        grid=(n // blk,),
        in_specs=[pl.BlockSpec((blk, h), lambda i: (i, 0)),
                  pl.BlockSpec((NC, blk, h), lambda i: (0, i, 0)),
                  pl.BlockSpec((1, h), lambda i: (0, 0)),
                  pl.BlockSpec((1, h), lambda i: (0, 0)),
                  pl.BlockSpec((1, 1), lambda i: (0, 0))],
        out_specs=pl.BlockSpec((blk, 1), lambda i: (i, 0)),
        out_shape=jax.ShapeDtypeStruct((n, 1), jnp.float32),
    )(p, agg, b1.reshape(1, h), w2.reshape(1, h), b2.reshape(1, 1))


def kernel(x, edge_index, W11, b11, W12, b12, W21, b21, W22, b22, W31, b31,
           W32, b32):
    n, d = x.shape
    h = W11.shape[1]
    e = edge_index.shape[1]

    cpw = -(-e // (NW * CHUNK))
    e_pad = cpw * NW * CHUNK
    acc_rows = NS * (-(-(n + 1) // NS))
    blk = 1000 if n % 1000 == 0 else 8 * (n // 8)

    src = edge_index[0]
    dst = edge_index[1]
    pad = e_pad - e
    src2d = jnp.concatenate(
        [src, jnp.zeros((pad,), jnp.int32)]).reshape(e_pad // CHUNK, CHUNK)
    dst2d = jnp.concatenate(
        [dst, jnp.full((pad,), n, jnp.int32)]).reshape(e_pad // CHUNK, CHUNK)
    zblk = jnp.zeros((acc_rows // NS, h), jnp.float32)

    m2 = W12 @ W21
    c2 = b12 @ W21
    m3 = W22 @ W31
    c3 = b22 @ W31

    a1 = _proj_first(x, W11, blk)
    g1 = _segment_sum_sc(a1, src2d, dst2d, zblk, n, acc_rows, cpw)
    a2 = _mid_stage(a1, g1, b11, m2, c2, blk)
    g2 = _segment_sum_sc(a2, src2d, dst2d, zblk, n, acc_rows, cpw)
    a3 = _mid_stage(a2, g2, b21, m3, c3, blk)
    g3 = _segment_sum_sc(a3, src2d, dst2d, zblk, n, acc_rows, cpw)
    return _final_stage(a3, g3, b31, W32, b32, blk)


# speedup vs baseline: 8.8021x; 8.8021x over previous
import functools

import jax
import jax.numpy as jnp
from jax import lax
from jax.experimental import pallas as pl
from jax.experimental.pallas import tpu as pltpu
from jax.experimental.pallas import tpu_sc as plsc

NC = 2
NS = 16
NW = NC * NS
CHUNK = 128


def _segment_sum_sc(p, src2d, dst2d, zblk, n_nodes, acc_rows, cpw):
    H = p.shape[1]
    sub_rows = acc_rows // NS
    mesh = plsc.VectorSubcoreMesh(core_axis_name="c", subcore_axis_name="s")
    last_rows = n_nodes - (NS - 1) * sub_rows

    @functools.partial(
        pl.kernel,
        out_type=jax.ShapeDtypeStruct((NC, n_nodes, H), jnp.float32),
        mesh=mesh,
        compiler_params=pltpu.CompilerParams(use_tc_tiling_on_sc=False),
        scratch_types=[
            pltpu.VMEM((cpw, CHUNK), jnp.int32),
            pltpu.VMEM((cpw, CHUNK), jnp.int32),
            pltpu.VMEM((CHUNK, H), jnp.float32),
            pltpu.VMEM_SHARED((acc_rows, H), jnp.float32),
        ],
    )
    def seg_kernel(p_hbm, s_hbm, d_hbm, z_hbm, out_hbm, sblk, dblk, rows, acc):
        c = lax.axis_index("c")
        s = lax.axis_index("s")
        w = c * NS + s

        pltpu.sync_copy(z_hbm, acc.at[pl.ds(s * sub_rows, sub_rows)])
        pltpu.sync_copy(s_hbm.at[w], sblk)
        pltpu.sync_copy(d_hbm.at[w], dblk)
        plsc.subcore_barrier()

        @pl.loop(0, cpw)
        def _(i):
            pltpu.sync_copy(p_hbm.at[sblk.at[i]], rows)
            pltpu.sync_copy(rows, acc.at[dblk.at[i]], add=True)

        plsc.subcore_barrier()

        @pl.when(s < NS - 1)
        def _():
            pltpu.sync_copy(acc.at[pl.ds(s * sub_rows, sub_rows)],
                            out_hbm.at[c, pl.ds(s * sub_rows, sub_rows)])

        @pl.when(s == NS - 1)
        def _():
            pltpu.sync_copy(acc.at[pl.ds((NS - 1) * sub_rows, last_rows)],
                            out_hbm.at[c, pl.ds((NS - 1) * sub_rows, last_rows)])

    return seg_kernel(p, src2d, dst2d, zblk)


def _proj_first(x, w, blk):
    n, d = x.shape
    h = w.shape[1]

    def body(x_ref, w_ref, o_ref):
        o_ref[...] = jnp.dot(x_ref[...], w_ref[...],
                             precision=lax.Precision.HIGHEST,
                             preferred_element_type=jnp.float32)

    return pl.pallas_call(
        body,
        grid=(n // blk,),
        in_specs=[pl.BlockSpec((blk, d), lambda i: (i, 0)),
                  pl.BlockSpec((d, h), lambda i: (0, 0))],
        out_specs=pl.BlockSpec((blk, h), lambda i: (i, 0)),
        out_shape=jax.ShapeDtypeStruct((n, h), jnp.float32),
    )(x, w)


def _mid_stage(p, agg, b1, m, c, blk):
    n, h = p.shape
    ho = m.shape[1]

    def body(p_ref, g_ref, b_ref, m_ref, c_ref, o_ref):
        u = jnp.maximum(p_ref[...] + g_ref[0] + g_ref[1] + b_ref[...], 0.0)
        o_ref[...] = jnp.dot(u, m_ref[...],
                             precision=lax.Precision.HIGHEST,
                             preferred_element_type=jnp.float32) + c_ref[...]

    return pl.pallas_call(
        body,
        grid=(n // blk,),
        in_specs=[pl.BlockSpec((blk, h), lambda i: (i, 0)),
                  pl.BlockSpec((NC, blk, h), lambda i: (0, i, 0)),
                  pl.BlockSpec((1, h), lambda i: (0, 0)),
                  pl.BlockSpec((h, ho), lambda i: (0, 0)),
                  pl.BlockSpec((1, ho), lambda i: (0, 0))],
        out_specs=pl.BlockSpec((blk, ho), lambda i: (i, 0)),
        out_shape=jax.ShapeDtypeStruct((n, ho), jnp.float32),
    )(p, agg, b1.reshape(1, h), m, c.reshape(1, ho))


def _final_stage(p, agg, b1, w2, b2, blk):
    n, h = p.shape

    def body(p_ref, g_ref, b_ref, w_ref, c_ref, o_ref):
        u = jnp.maximum(p_ref[...] + g_ref[0] + g_ref[1] + b_ref[...], 0.0)
        o_ref[...] = jnp.sum(u * w_ref[...], axis=1, keepdims=True) + c_ref[...]

    return pl.pallas_call(
        body,
        grid=(n // blk,),
        in_specs=[pl.BlockSpec((blk, h), lambda i: (i, 0)),
                  pl.BlockSpec((NC, blk, h), lambda i: (0, i, 0)),
                  pl.BlockSpec((1, h), lambda i: (0, 0)),
                  pl.BlockSpec((1, h), lambda i: (0, 0)),
                  pl.BlockSpec((1, 1), lambda i: (0, 0))],
        out_specs=pl.BlockSpec((blk, 1), lambda i: (i, 0)),
        out_shape=jax.ShapeDtypeStruct((n, 1), jnp.float32),
    )(p, agg, b1.reshape(1, h), w2.reshape(1, h), b2.reshape(1, 1))


def kernel(x, edge_index, W11, b11, W12, b12, W21, b21, W22, b22, W31, b31,
           W32, b32):
    n, d = x.shape
    h = W11.shape[1]
    e = edge_index.shape[1]

    cpw = -(-e // (NW * CHUNK))
    e_pad = cpw * NW * CHUNK
    sub_rows = 8 * (-(-(n + 1) // (NS * 8)))
    acc_rows = NS * sub_rows
    blk = 1000 if n % 1000 == 0 else 8 * (n // 8)

    src = edge_index[0]
    dst = edge_index[1]
    pad = e_pad - e
    src2d = jnp.concatenate(
        [src, jnp.zeros((pad,), jnp.int32)]).reshape(NW, cpw, CHUNK)
    dst2d = jnp.concatenate(
        [dst, jnp.full((pad,), n, jnp.int32)]).reshape(NW, cpw, CHUNK)
    zblk = jnp.zeros((acc_rows // NS, h), jnp.float32)

    hp = lax.Precision.HIGHEST
    m2 = jnp.dot(W12, W21, precision=hp)
    c2 = jnp.dot(b12, W21, precision=hp)
    m3 = jnp.dot(W22, W31, precision=hp)
    c3 = jnp.dot(b22, W31, precision=hp)

    a1 = _proj_first(x, W11, blk)
    g1 = _segment_sum_sc(a1, src2d, dst2d, zblk, n, acc_rows, cpw)
    a2 = _mid_stage(a1, g1, b11, m2, c2, blk)
    g2 = _segment_sum_sc(a2, src2d, dst2d, zblk, n, acc_rows, cpw)
    a3 = _mid_stage(a2, g2, b21, m3, c3, blk)
    g3 = _segment_sum_sc(a3, src2d, dst2d, zblk, n, acc_rows, cpw)
    return _final_stage(a3, g3, b31, W32, b32, blk)


# speedup vs baseline: 15.2802x; 1.7360x over previous
import functools

import jax
import jax.numpy as jnp
from jax import lax
from jax.experimental import pallas as pl
from jax.experimental.pallas import tpu as pltpu
from jax.experimental.pallas import tpu_sc as plsc

NC = 2
NS = 16
NW = NC * NS
CHUNK = 128


def _segment_sum_sc(p, src2d, dst2d, zblk, n_nodes, acc_rows, cpw):
    H = p.shape[1]
    sub_rows = acc_rows // NS
    mesh = plsc.VectorSubcoreMesh(core_axis_name="c", subcore_axis_name="s")
    last_rows = n_nodes - (NS - 1) * sub_rows
    NB = 4

    @functools.partial(
        pl.kernel,
        out_type=jax.ShapeDtypeStruct((NC, n_nodes, H), jnp.float32),
        mesh=mesh,
        compiler_params=pltpu.CompilerParams(use_tc_tiling_on_sc=False),
        scratch_types=[
            pltpu.VMEM((cpw, CHUNK), jnp.int32),
            pltpu.VMEM((cpw, CHUNK), jnp.int32),
            pltpu.VMEM((NB, CHUNK, H), jnp.float32),
            pltpu.VMEM_SHARED((n_nodes, H), jnp.float32),
            pltpu.VMEM_SHARED((acc_rows, H), jnp.float32),
        ] + [pltpu.SemaphoreType.DMA] * (2 * NB),
    )
    def seg_kernel(p_hbm, s_hbm, d_hbm, z_hbm, out_hbm, sblk, dblk, rows,
                   table, acc, *sems):
        gs = sems[:NB]
        ss = sems[NB:]
        c = lax.axis_index("c")
        s = lax.axis_index("s")
        w = c * NS + s

        pltpu.sync_copy(z_hbm, acc.at[pl.ds(s * sub_rows, sub_rows)])

        @pl.when(s < NS - 1)
        def _():
            pltpu.sync_copy(p_hbm.at[pl.ds(s * sub_rows, sub_rows)],
                            table.at[pl.ds(s * sub_rows, sub_rows)])

        @pl.when(s == NS - 1)
        def _():
            pltpu.sync_copy(p_hbm.at[pl.ds((NS - 1) * sub_rows, last_rows)],
                            table.at[pl.ds((NS - 1) * sub_rows, last_rows)])

        pltpu.sync_copy(s_hbm.at[w], sblk)
        pltpu.sync_copy(d_hbm.at[w], dblk)
        plsc.subcore_barrier()

        for j in range(NB):
            pltpu.async_copy(table.at[sblk.at[j]], rows.at[j], gs[j])

        @pl.loop(0, cpw, step=NB)
        def _(i):
            for j in range(NB):
                pltpu.make_async_copy(table.at[sblk.at[0]],
                                      rows.at[j], gs[j]).wait()
                pltpu.async_copy(rows.at[j], acc.at[dblk.at[i + j]], ss[j],
                                 add=True)
            for j in range(NB):
                @pl.when(i + NB + j < cpw)
                def _():
                    pltpu.make_async_copy(rows.at[j], acc.at[dblk.at[0]],
                                          ss[j]).wait()
                    pltpu.async_copy(table.at[sblk.at[i + NB + j]],
                                     rows.at[j], gs[j])

        for j in range(NB):
            pltpu.make_async_copy(rows.at[j], acc.at[dblk.at[0]], ss[j]).wait()

        plsc.subcore_barrier()

        @pl.when(s < NS - 1)
        def _():
            pltpu.sync_copy(acc.at[pl.ds(s * sub_rows, sub_rows)],
                            out_hbm.at[c, pl.ds(s * sub_rows, sub_rows)])

        @pl.when(s == NS - 1)
        def _():
            pltpu.sync_copy(acc.at[pl.ds((NS - 1) * sub_rows, last_rows)],
                            out_hbm.at[c, pl.ds((NS - 1) * sub_rows, last_rows)])

    return seg_kernel(p, src2d, dst2d, zblk)


def _proj_first(x, w, blk):
    n, d = x.shape
    h = w.shape[1]

    def body(x_ref, w_ref, o_ref):
        o_ref[...] = jnp.dot(x_ref[...], w_ref[...],
                             precision=lax.Precision.HIGHEST,
                             preferred_element_type=jnp.float32)

    return pl.pallas_call(
        body,
        grid=(n // blk,),
        in_specs=[pl.BlockSpec((blk, d), lambda i: (i, 0)),
                  pl.BlockSpec((d, h), lambda i: (0, 0))],
        out_specs=pl.BlockSpec((blk, h), lambda i: (i, 0)),
        out_shape=jax.ShapeDtypeStruct((n, h), jnp.float32),
    )(x, w)


def _mid_stage(p, agg, b1, m, c, blk):
    n, h = p.shape
    ho = m.shape[1]

    def body(p_ref, g_ref, b_ref, m_ref, c_ref, o_ref):
        u = jnp.maximum(p_ref[...] + g_ref[0] + g_ref[1] + b_ref[...], 0.0)
        o_ref[...] = jnp.dot(u, m_ref[...],
                             precision=lax.Precision.HIGHEST,
                             preferred_element_type=jnp.float32) + c_ref[...]

    return pl.pallas_call(
        body,
        grid=(n // blk,),
        in_specs=[pl.BlockSpec((blk, h), lambda i: (i, 0)),
                  pl.BlockSpec((NC, blk, h), lambda i: (0, i, 0)),
                  pl.BlockSpec((1, h), lambda i: (0, 0)),
                  pl.BlockSpec((h, ho), lambda i: (0, 0)),
                  pl.BlockSpec((1, ho), lambda i: (0, 0))],
        out_specs=pl.BlockSpec((blk, ho), lambda i: (i, 0)),
        out_shape=jax.ShapeDtypeStruct((n, ho), jnp.float32),
    )(p, agg, b1.reshape(1, h), m, c.reshape(1, ho))


def _final_stage(p, agg, b1, w2, b2, blk):
    n, h = p.shape

    def body(p_ref, g_ref, b_ref, w_ref, c_ref, o_ref):
        u = jnp.maximum(p_ref[...] + g_ref[0] + g_ref[1] + b_ref[...], 0.0)
        o_ref[...] = jnp.sum(u * w_ref[...], axis=1, keepdims=True) + c_ref[...]

    return pl.pallas_call(
        body,
        grid=(n // blk,),
        in_specs=[pl.BlockSpec((blk, h), lambda i: (i, 0)),
                  pl.BlockSpec((NC, blk, h), lambda i: (0, i, 0)),
                  pl.BlockSpec((1, h), lambda i: (0, 0)),
                  pl.BlockSpec((1, h), lambda i: (0, 0)),
                  pl.BlockSpec((1, 1), lambda i: (0, 0))],
        out_specs=pl.BlockSpec((blk, 1), lambda i: (i, 0)),
        out_shape=jax.ShapeDtypeStruct((n, 1), jnp.float32),
    )(p, agg, b1.reshape(1, h), w2.reshape(1, h), b2.reshape(1, 1))


def kernel(x, edge_index, W11, b11, W12, b12, W21, b21, W22, b22, W31, b31,
           W32, b32):
    n, d = x.shape
    h = W11.shape[1]
    e = edge_index.shape[1]

    cpw = 4 * (-(-e // (NW * CHUNK * 4)))
    e_pad = cpw * NW * CHUNK
    sub_rows = 8 * (-(-(n + 1) // (NS * 8)))
    acc_rows = NS * sub_rows
    blk = 1000 if n % 1000 == 0 else 8 * (n // 8)

    src = edge_index[0]
    dst = edge_index[1]
    pad = e_pad - e
    src2d = jnp.concatenate(
        [src, jnp.zeros((pad,), jnp.int32)]).reshape(NW, cpw, CHUNK)
    dst2d = jnp.concatenate(
        [dst, jnp.full((pad,), n, jnp.int32)]).reshape(NW, cpw, CHUNK)
    zblk = jnp.zeros((acc_rows // NS, h), jnp.float32)

    hp = lax.Precision.HIGHEST
    m2 = jnp.dot(W12, W21, precision=hp)
    c2 = jnp.dot(b12, W21, precision=hp)
    m3 = jnp.dot(W22, W31, precision=hp)
    c3 = jnp.dot(b22, W31, precision=hp)

    a1 = _proj_first(x, W11, blk)
    g1 = _segment_sum_sc(a1, src2d, dst2d, zblk, n, acc_rows, cpw)
    a2 = _mid_stage(a1, g1, b11, m2, c2, blk)
    g2 = _segment_sum_sc(a2, src2d, dst2d, zblk, n, acc_rows, cpw)
    a3 = _mid_stage(a2, g2, b21, m3, c3, blk)
    g3 = _segment_sum_sc(a3, src2d, dst2d, zblk, n, acc_rows, cpw)
    return _final_stage(a3, g3, b31, W32, b32, blk)


# speedup vs baseline: 16.6692x; 1.0909x over previous
import functools

import jax
import jax.numpy as jnp
from jax import lax
from jax.experimental import pallas as pl
from jax.experimental.pallas import tpu as pltpu
from jax.experimental.pallas import tpu_sc as plsc

NC = 2
NS = 16
NW = NC * NS
CHUNK = 128


def _segment_sum_sc(p, eip, zblk, n_nodes, acc_rows, cpw):
    H = p.shape[1]
    sub_rows = acc_rows // NS
    mesh = plsc.VectorSubcoreMesh(core_axis_name="c", subcore_axis_name="s")
    last_rows = n_nodes - (NS - 1) * sub_rows
    NB = 8

    @functools.partial(
        pl.kernel,
        out_type=jax.ShapeDtypeStruct((NC, n_nodes, H), jnp.float32),
        mesh=mesh,
        compiler_params=pltpu.CompilerParams(use_tc_tiling_on_sc=False),
        scratch_types=[
            pltpu.VMEM((cpw, CHUNK), jnp.int32),
            pltpu.VMEM((cpw, CHUNK), jnp.int32),
            pltpu.VMEM((NB, CHUNK, H), jnp.float32),
            pltpu.VMEM_SHARED((acc_rows, H), jnp.float32),
            pltpu.VMEM_SHARED((acc_rows, H), jnp.float32),
        ] + [pltpu.SemaphoreType.DMA] * (2 * NB),
    )
    def seg_kernel(p_hbm, e_hbm, z_hbm, out_hbm, sblk, dblk, rows,
                   table, acc, *sems):
        gs = sems[:NB]
        ss = sems[NB:]
        c = lax.axis_index("c")
        s = lax.axis_index("s")
        w = c * NS + s

        pltpu.sync_copy(z_hbm, acc.at[pl.ds(s * sub_rows, sub_rows)])

        @pl.when(s < NS - 1)
        def _():
            pltpu.sync_copy(p_hbm.at[pl.ds(s * sub_rows, sub_rows)],
                            table.at[pl.ds(s * sub_rows, sub_rows)])

        @pl.when(s == NS - 1)
        def _():
            pltpu.sync_copy(p_hbm.at[pl.ds((NS - 1) * sub_rows, last_rows)],
                            table.at[pl.ds((NS - 1) * sub_rows, last_rows)])
            pltpu.sync_copy(z_hbm.at[pl.ds(0, acc_rows - n_nodes)],
                            table.at[pl.ds(n_nodes, acc_rows - n_nodes)])

        pltpu.sync_copy(e_hbm.at[0, w], sblk)
        pltpu.sync_copy(e_hbm.at[1, w], dblk)
        plsc.subcore_barrier()

        for j in range(NB):
            pltpu.async_copy(table.at[sblk.at[j]], rows.at[j], gs[j])

        @pl.loop(0, cpw, step=NB)
        def _(i):
            for j in range(NB):
                pltpu.make_async_copy(table.at[sblk.at[0]],
                                      rows.at[j], gs[j]).wait()
                pltpu.async_copy(rows.at[j], acc.at[dblk.at[i + j]], ss[j],
                                 add=True)
            for j in range(NB):
                @pl.when(i + NB + j < cpw)
                def _():
                    pltpu.make_async_copy(rows.at[j], acc.at[dblk.at[0]],
                                          ss[j]).wait()
                    pltpu.async_copy(table.at[sblk.at[i + NB + j]],
                                     rows.at[j], gs[j])

        for j in range(NB):
            pltpu.make_async_copy(rows.at[j], acc.at[dblk.at[0]], ss[j]).wait()

        plsc.subcore_barrier()

        @pl.when(s < NS - 1)
        def _():
            pltpu.sync_copy(acc.at[pl.ds(s * sub_rows, sub_rows)],
                            out_hbm.at[c, pl.ds(s * sub_rows, sub_rows)])

        @pl.when(s == NS - 1)
        def _():
            pltpu.sync_copy(acc.at[pl.ds((NS - 1) * sub_rows, last_rows)],
                            out_hbm.at[c, pl.ds((NS - 1) * sub_rows, last_rows)])

    return seg_kernel(p, eip, zblk)


def _proj_first(x, w, blk):
    n, d = x.shape
    h = w.shape[1]

    def body(x_ref, w_ref, o_ref):
        o_ref[...] = jnp.dot(x_ref[...], w_ref[...],
                             precision=lax.Precision.HIGHEST,
                             preferred_element_type=jnp.float32)

    return pl.pallas_call(
        body,
        grid=(n // blk,),
        in_specs=[pl.BlockSpec((blk, d), lambda i: (i, 0)),
                  pl.BlockSpec((d, h), lambda i: (0, 0))],
        out_specs=pl.BlockSpec((blk, h), lambda i: (i, 0)),
        out_shape=jax.ShapeDtypeStruct((n, h), jnp.float32),
    )(x, w)


def _mid_stage(p, agg, b1, m, c, blk):
    n, h = p.shape
    ho = m.shape[1]

    def body(p_ref, g_ref, b_ref, m_ref, c_ref, o_ref):
        u = jnp.maximum(p_ref[...] + g_ref[0] + g_ref[1] + b_ref[...], 0.0)
        o_ref[...] = jnp.dot(u, m_ref[...],
                             precision=lax.Precision.HIGHEST,
                             preferred_element_type=jnp.float32) + c_ref[...]

    return pl.pallas_call(
        body,
        grid=(n // blk,),
        in_specs=[pl.BlockSpec((blk, h), lambda i: (i, 0)),
                  pl.BlockSpec((NC, blk, h), lambda i: (0, i, 0)),
                  pl.BlockSpec((1, h), lambda i: (0, 0)),
                  pl.BlockSpec((h, ho), lambda i: (0, 0)),
                  pl.BlockSpec((1, ho), lambda i: (0, 0))],
        out_specs=pl.BlockSpec((blk, ho), lambda i: (i, 0)),
        out_shape=jax.ShapeDtypeStruct((n, ho), jnp.float32),
    )(p, agg, b1.reshape(1, h), m, c.reshape(1, ho))


def _final_stage(p, agg, b1, w2, b2, blk):
    n, h = p.shape

    def body(p_ref, g_ref, b_ref, w_ref, c_ref, o_ref):
        u = jnp.maximum(p_ref[...] + g_ref[0] + g_ref[1] + b_ref[...], 0.0)
        o_ref[...] = jnp.sum(u * w_ref[...], axis=1, keepdims=True) + c_ref[...]

    return pl.pallas_call(
        body,
        grid=(n // blk,),
        in_specs=[pl.BlockSpec((blk, h), lambda i: (i, 0)),
                  pl.BlockSpec((NC, blk, h), lambda i: (0, i, 0)),
                  pl.BlockSpec((1, h), lambda i: (0, 0)),
                  pl.BlockSpec((1, h), lambda i: (0, 0)),
                  pl.BlockSpec((1, 1), lambda i: (0, 0))],
        out_specs=pl.BlockSpec((blk, 1), lambda i: (i, 0)),
        out_shape=jax.ShapeDtypeStruct((n, 1), jnp.float32),
    )(p, agg, b1.reshape(1, h), w2.reshape(1, h), b2.reshape(1, 1))


def kernel(x, edge_index, W11, b11, W12, b12, W21, b21, W22, b22, W31, b31,
           W32, b32):
    n, d = x.shape
    h = W11.shape[1]
    e = edge_index.shape[1]

    cpw = 8 * (-(-e // (NW * CHUNK * 8)))
    e_pad = cpw * NW * CHUNK
    sub_rows = 8 * (-(-(n + 1) // (NS * 8)))
    acc_rows = NS * sub_rows
    blk = 1000 if n % 1000 == 0 else 8 * (n // 8)

    pad = e_pad - e
    eip = jnp.pad(edge_index, ((0, 0), (0, pad)),
                  constant_values=n).reshape(2, NW, cpw, CHUNK)
    zblk = jnp.zeros((acc_rows // NS, h), jnp.float32)

    hp = lax.Precision.HIGHEST
    m2 = jnp.dot(W12, W21, precision=hp)
    c2 = jnp.dot(b12, W21, precision=hp)
    m3 = jnp.dot(W22, W31, precision=hp)
    c3 = jnp.dot(b22, W31, precision=hp)

    a1 = _proj_first(x, W11, blk)
    g1 = _segment_sum_sc(a1, eip, zblk, n, acc_rows, cpw)
    a2 = _mid_stage(a1, g1, b11, m2, c2, blk)
    g2 = _segment_sum_sc(a2, eip, zblk, n, acc_rows, cpw)
    a3 = _mid_stage(a2, g2, b21, m3, c3, blk)
    g3 = _segment_sum_sc(a3, eip, zblk, n, acc_rows, cpw)
    return _final_stage(a3, g3, b31, W32, b32, blk)
